# Initial kernel scaffold; baseline (speedup 1.0000x reference)
import jax
import jax.numpy as jnp
from jax.experimental import pallas as pl


def kernel(tokens, vocab_map):
    raise NotImplementedError("write your pallas kernel here")


# speedup vs baseline: 1.4794x; 1.4794x over previous
import functools

import jax
import jax.numpy as jnp
from jax import lax
from jax.experimental import pallas as pl
from jax.experimental.pallas import tpu as pltpu
from jax.experimental.pallas import tpu_sc as plsc

_NUM_CORES = 2
_NUM_SUBCORES = 16
_NUM_WORKERS = _NUM_CORES * _NUM_SUBCORES


@functools.lru_cache(maxsize=None)
def _make_gather(n):
    assert n % (8 * _NUM_WORKERS) == 0
    b_per_w = n // _NUM_WORKERS
    mesh = plsc.VectorSubcoreMesh(core_axis_name="c", subcore_axis_name="s")

    @functools.partial(
        pl.kernel,
        mesh=mesh,
        out_type=jax.ShapeDtypeStruct((n,), jnp.int32),
        scratch_types=[
            pltpu.VMEM((b_per_w,), jnp.int32),
            pltpu.VMEM((b_per_w,), jnp.int32),
            pltpu.SemaphoreType.DMA,
        ],
    )
    def k(vocab_hbm, tok_hbm, out_hbm, idx_v, rows_v, sem):
        wid = lax.axis_index("s") * _NUM_CORES + lax.axis_index("c")
        base = wid * b_per_w
        pltpu.sync_copy(tok_hbm.at[pl.ds(base, b_per_w)], idx_v)
        pltpu.async_copy(vocab_hbm.at[idx_v], rows_v, sem).wait()
        pltpu.sync_copy(rows_v, out_hbm.at[pl.ds(base, b_per_w)])

    return k


def kernel(tokens, vocab_map):
    flat = tokens.reshape(-1)
    out = _make_gather(flat.shape[0])(vocab_map, flat)
    return out.reshape(tokens.shape)


# speedup vs baseline: 1.8384x; 1.2426x over previous
import functools

import jax
import jax.numpy as jnp
from jax import lax
from jax.experimental import pallas as pl
from jax.experimental.pallas import tpu as pltpu
from jax.experimental.pallas import tpu_sc as plsc

_NUM_CORES = 2
_NUM_SUBCORES = 16
_NUM_WORKERS = _NUM_CORES * _NUM_SUBCORES


@functools.lru_cache(maxsize=None)
def _make_gather(n, vocab):
    assert n % (8 * _NUM_WORKERS) == 0
    b_per_w = n // _NUM_WORKERS
    bounds = [(i * vocab // _NUM_SUBCORES) // 8 * 8 for i in range(_NUM_SUBCORES)]
    bounds.append(vocab)
    mesh = plsc.VectorSubcoreMesh(core_axis_name="c", subcore_axis_name="s")

    @functools.partial(
        pl.kernel,
        mesh=mesh,
        out_type=jax.ShapeDtypeStruct((n,), jnp.int32),
        scratch_types=[
            pltpu.VMEM_SHARED((vocab,), jnp.int32),
            pltpu.VMEM((b_per_w,), jnp.int32),
            pltpu.VMEM((b_per_w,), jnp.int32),
            pltpu.SemaphoreType.DMA,
        ],
    )
    def k(vocab_hbm, tok_hbm, out_hbm, table_sh, idx_v, rows_v, sem):
        sid = lax.axis_index("s")
        wid = sid * _NUM_CORES + lax.axis_index("c")
        base = wid * b_per_w
        for i in range(_NUM_SUBCORES):
            lo, hi = bounds[i], bounds[i + 1]

            @pl.when(sid == i)
            def _():
                p, j = lo, 0
                while p < hi:
                    sz = min(b_per_w, hi - p)
                    buf = (idx_v, rows_v)[j % 2]
                    pltpu.sync_copy(vocab_hbm.at[pl.ds(p, sz)], buf.at[pl.ds(0, sz)])
                    pltpu.sync_copy(buf.at[pl.ds(0, sz)], table_sh.at[pl.ds(p, sz)])
                    p += sz
                    j += 1

        pltpu.sync_copy(tok_hbm.at[pl.ds(base, b_per_w)], idx_v)
        plsc.subcore_barrier()
        pltpu.async_copy(table_sh.at[idx_v], rows_v, sem).wait()
        pltpu.sync_copy(rows_v, out_hbm.at[pl.ds(base, b_per_w)])

    return k


def kernel(tokens, vocab_map):
    flat = tokens.reshape(-1)
    out = _make_gather(flat.shape[0], vocab_map.shape[0])(vocab_map, flat)
    return out.reshape(tokens.shape)
